# baseline (speedup 1.0000x reference)
import functools

import jax
import jax.numpy as jnp
from jax import lax
from jax.experimental import pallas as pl
from jax.experimental.pallas import tpu as pltpu
from jax.experimental.pallas import tpu_sc as plsc

_B = 4096
_N = 100000
_DIM = 64
_VOLUME_TEMP = 1.0
_INTERSECTION_TEMP = 0.01
_SOFTPLUS_CONST = 2 * _INTERSECTION_TEMP * 0.5772156649015329

_info = plsc.get_sparse_core_info()
_NC, _NS = _info.num_cores, _info.num_subcores
_NW = _NC * _NS
_BPW = _B // _NW


_sc_mesh = plsc.VectorSubcoreMesh(core_axis_name="c", subcore_axis_name="s")


@functools.partial(
    pl.kernel,
    mesh=_sc_mesh,
    out_type=[jax.ShapeDtypeStruct((_B, _DIM), jnp.float32)] * 4,
    scratch_types=[
        pltpu.VMEM((_BPW,), jnp.int32),
        pltpu.VMEM((_BPW,), jnp.int32),
        pltpu.VMEM((_BPW, _DIM), jnp.float32),
        pltpu.VMEM((_BPW, _DIM), jnp.float32),
        pltpu.VMEM((_BPW, _DIM), jnp.float32),
        pltpu.VMEM((_BPW, _DIM), jnp.float32),
        pltpu.SemaphoreType.DMA,
    ],
    compiler_params=pltpu.CompilerParams(use_tc_tiling_on_sc=False),
)
def _sc_gather(t1_hbm, t2_hbm, i1_hbm, i2_hbm,
               o11, o12, o21, o22,
               idx1_v, idx2_v, r11_v, r12_v, r21_v, r22_v, sem):
    wid = lax.axis_index("s") * _NC + lax.axis_index("c")
    base = wid * _BPW
    pltpu.sync_copy(i1_hbm.at[pl.ds(base, _BPW)], idx1_v)
    pltpu.sync_copy(i2_hbm.at[pl.ds(base, _BPW)], idx2_v)
    d1 = pltpu.async_copy(t1_hbm.at[idx1_v], r11_v, sem)
    d2 = pltpu.async_copy(t2_hbm.at[idx1_v], r12_v, sem)
    d3 = pltpu.async_copy(t1_hbm.at[idx2_v], r21_v, sem)
    d4 = pltpu.async_copy(t2_hbm.at[idx2_v], r22_v, sem)
    d1.wait()
    d2.wait()
    d3.wait()
    d4.wait()
    pltpu.sync_copy(r11_v, o11.at[pl.ds(base, _BPW)])
    pltpu.sync_copy(r12_v, o12.at[pl.ds(base, _BPW)])
    pltpu.sync_copy(r21_v, o21.at[pl.ds(base, _BPW)])
    pltpu.sync_copy(r22_v, o22.at[pl.ds(base, _BPW)])


def _softplus(x):
    return jnp.logaddexp(x, 0.0)


def _box_math_body(c1_ref, e1_ref, c2_ref, e2_ref, out_ref):
    c1 = c1_ref[...]
    w1 = _softplus(e1_ref[...]) * 0.5
    c2 = c2_ref[...]
    w2 = _softplus(e2_ref[...]) * 0.5
    min1 = c1 - w1
    max1 = c1 + w1
    min2 = c2 - w2
    max2 = c2 + w2
    t = _INTERSECTION_TEMP
    meet_min = t * jnp.logaddexp(min1 / t, min2 / t)
    meet_max = -t * jnp.logaddexp(-max1 / t, -max2 / t)
    meet_min = jnp.maximum(meet_min, jnp.maximum(min1, min2))
    meet_max = jnp.minimum(meet_max, jnp.minimum(max1, max2))
    log_overlap = jnp.sum(
        jnp.log(_VOLUME_TEMP * _softplus(
            (meet_max - meet_min - _SOFTPLUS_CONST) / _VOLUME_TEMP) + 1e-20),
        axis=-1)
    log_rhs = jnp.sum(
        jnp.log(_VOLUME_TEMP * _softplus(
            (max2 - min2 - _SOFTPLUS_CONST) / _VOLUME_TEMP) + 1e-20),
        axis=-1)
    out_ref[...] = log_overlap - log_rhs


_TC_BLOCK = 512


def _tc_math(r11, r12, r21, r22):
    grid = _B // _TC_BLOCK
    in_spec = pl.BlockSpec((_TC_BLOCK, _DIM), lambda i: (i, 0))
    return pl.pallas_call(
        _box_math_body,
        grid=(grid,),
        in_specs=[in_spec] * 4,
        out_specs=pl.BlockSpec((_TC_BLOCK,), lambda i: (i,)),
        out_shape=jax.ShapeDtypeStruct((_B,), jnp.float32),
    )(r11, r12, r21, r22)


def kernel(idx1, idx2, emb1, emb2, embs1_weight, embs2_weight):
    del emb1, emb2
    i1 = idx1.astype(jnp.int32)
    i2 = idx2.astype(jnp.int32)
    r11, r12, r21, r22 = _sc_gather(embs1_weight, embs2_weight, i1, i2)
    return _tc_math(r11, r12, r21, r22)


# speedup vs baseline: 1.0254x; 1.0254x over previous
import functools

import jax
import jax.numpy as jnp
from jax import lax
from jax.experimental import pallas as pl
from jax.experimental.pallas import tpu as pltpu
from jax.experimental.pallas import tpu_sc as plsc

_B = 4096
_N = 100000
_DIM = 64
_VOLUME_TEMP = 1.0
_INTERSECTION_TEMP = 0.01
_SOFTPLUS_CONST = 2 * _INTERSECTION_TEMP * 0.5772156649015329

_info = plsc.get_sparse_core_info()
_NC, _NS, _NL = _info.num_cores, _info.num_subcores, _info.num_lanes
_NW = _NC * _NS
_BPW = _B // _NW


_sc_mesh = plsc.VectorSubcoreMesh(core_axis_name="c", subcore_axis_name="s")


@functools.partial(
    pl.kernel,
    mesh=_sc_mesh,
    out_type=[jax.ShapeDtypeStruct((_B, 2 * _DIM), jnp.float32)] * 4,
    scratch_types=[
        pltpu.VMEM((_BPW,), jnp.int32),
        pltpu.VMEM((_BPW,), jnp.int32),
        pltpu.VMEM((_BPW, 2 * _DIM), jnp.float32),
        pltpu.VMEM((_BPW, 2 * _DIM), jnp.float32),
        pltpu.VMEM((_BPW, 2 * _DIM), jnp.float32),
        pltpu.VMEM((_BPW, 2 * _DIM), jnp.float32),
        pltpu.SemaphoreType.DMA,
    ],
)
def _sc_gather(t1_hbm, t2_hbm, i1_hbm, i2_hbm,
               o11, o12, o21, o22,
               idx1_v, idx2_v, r11_v, r12_v, r21_v, r22_v, sem):
    wid = lax.axis_index("s") * _NC + lax.axis_index("c")
    base = wid * _BPW
    pltpu.sync_copy(i1_hbm.at[pl.ds(base, _BPW)], idx1_v)
    pltpu.sync_copy(i2_hbm.at[pl.ds(base, _BPW)], idx2_v)
    for g in range(_BPW // _NL):
        sl = pl.ds(g * _NL, _NL)
        idx1_v[sl] = jax.lax.shift_right_logical(idx1_v[sl], 1)
        idx2_v[sl] = jax.lax.shift_right_logical(idx2_v[sl], 1)
    d1 = pltpu.async_copy(t1_hbm.at[idx1_v], r11_v, sem)
    d2 = pltpu.async_copy(t2_hbm.at[idx1_v], r12_v, sem)
    d3 = pltpu.async_copy(t1_hbm.at[idx2_v], r21_v, sem)
    d4 = pltpu.async_copy(t2_hbm.at[idx2_v], r22_v, sem)
    d1.wait()
    d2.wait()
    d3.wait()
    d4.wait()
    pltpu.sync_copy(r11_v, o11.at[pl.ds(base, _BPW)])
    pltpu.sync_copy(r12_v, o12.at[pl.ds(base, _BPW)])
    pltpu.sync_copy(r21_v, o21.at[pl.ds(base, _BPW)])
    pltpu.sync_copy(r22_v, o22.at[pl.ds(base, _BPW)])


def _softplus(x):
    return jnp.logaddexp(x, 0.0)


def _half(ref, parity):
    x = ref[...]
    return jnp.where(parity, x[:, _DIM:], x[:, :_DIM])


def _box_math_body(i1_ref, i2_ref, c1_ref, e1_ref, c2_ref, e2_ref, out_ref):
    p1 = (i1_ref[...] & 1)[:, None] == 1
    p2 = (i2_ref[...] & 1)[:, None] == 1
    c1 = _half(c1_ref, p1)
    w1 = _softplus(_half(e1_ref, p1)) * 0.5
    c2 = _half(c2_ref, p2)
    w2 = _softplus(_half(e2_ref, p2)) * 0.5
    min1 = c1 - w1
    max1 = c1 + w1
    min2 = c2 - w2
    max2 = c2 + w2
    t = _INTERSECTION_TEMP
    meet_min = t * jnp.logaddexp(min1 / t, min2 / t)
    meet_max = -t * jnp.logaddexp(-max1 / t, -max2 / t)
    meet_min = jnp.maximum(meet_min, jnp.maximum(min1, min2))
    meet_max = jnp.minimum(meet_max, jnp.minimum(max1, max2))
    log_overlap = jnp.sum(
        jnp.log(_VOLUME_TEMP * _softplus(
            (meet_max - meet_min - _SOFTPLUS_CONST) / _VOLUME_TEMP) + 1e-20),
        axis=-1)
    log_rhs = jnp.sum(
        jnp.log(_VOLUME_TEMP * _softplus(
            (max2 - min2 - _SOFTPLUS_CONST) / _VOLUME_TEMP) + 1e-20),
        axis=-1)
    out_ref[...] = log_overlap - log_rhs


_TC_BLOCK = 512


def _tc_math(i1, i2, r11, r12, r21, r22):
    grid = _B // _TC_BLOCK
    in_spec = pl.BlockSpec((_TC_BLOCK, 2 * _DIM), lambda i: (i, 0))
    idx_spec = pl.BlockSpec((_TC_BLOCK,), lambda i: (i,))
    return pl.pallas_call(
        _box_math_body,
        grid=(grid,),
        in_specs=[idx_spec, idx_spec] + [in_spec] * 4,
        out_specs=pl.BlockSpec((_TC_BLOCK,), lambda i: (i,)),
        out_shape=jax.ShapeDtypeStruct((_B,), jnp.float32),
    )(i1, i2, r11, r12, r21, r22)


def kernel(idx1, idx2, emb1, emb2, embs1_weight, embs2_weight):
    del emb1, emb2
    i1 = idx1.astype(jnp.int32)
    i2 = idx2.astype(jnp.int32)
    t1p = jnp.reshape(embs1_weight, (_N // 2, 2 * _DIM))
    t2p = jnp.reshape(embs2_weight, (_N // 2, 2 * _DIM))
    r11, r12, r21, r22 = _sc_gather(t1p, t2p, i1, i2)
    return _tc_math(i1, i2, r11, r12, r21, r22)


# speedup vs baseline: 1.2916x; 1.2596x over previous
import functools

import jax
import jax.numpy as jnp
from jax import lax
from jax.experimental import pallas as pl
from jax.experimental.pallas import tpu as pltpu
from jax.experimental.pallas import tpu_sc as plsc

_B = 4096
_N = 100000
_DIM = 64
_VOLUME_TEMP = 1.0
_INTERSECTION_TEMP = 0.01
_SOFTPLUS_CONST = 2 * _INTERSECTION_TEMP * 0.5772156649015329

_info = plsc.get_sparse_core_info()
_NC, _NS, _NL = _info.num_cores, _info.num_subcores, _info.num_lanes
_NW = _NC * _NS
_BPW = _B // _NW


_sc_mesh = plsc.VectorSubcoreMesh(core_axis_name="c", subcore_axis_name="s")


@functools.partial(
    pl.kernel,
    mesh=_sc_mesh,
    out_type=[jax.ShapeDtypeStruct((_B, 2 * _DIM), jnp.float32)] * 4,
    scratch_types=[
        pltpu.VMEM((_BPW,), jnp.int32),
        pltpu.VMEM((_BPW,), jnp.int32),
        pltpu.VMEM((_BPW, 2 * _DIM), jnp.float32),
        pltpu.VMEM((_BPW, 2 * _DIM), jnp.float32),
        pltpu.VMEM((_BPW, 2 * _DIM), jnp.float32),
        pltpu.VMEM((_BPW, 2 * _DIM), jnp.float32),
        pltpu.SemaphoreType.DMA,
    ],
)
def _sc_gather(t1_hbm, t2_hbm, i1_hbm, i2_hbm,
               o11, o12, o21, o22,
               idx1_v, idx2_v, r11_v, r12_v, r21_v, r22_v, sem):
    wid = lax.axis_index("s") * _NC + lax.axis_index("c")
    base = wid * _BPW
    pltpu.sync_copy(i1_hbm.at[pl.ds(base, _BPW)], idx1_v)
    pltpu.sync_copy(i2_hbm.at[pl.ds(base, _BPW)], idx2_v)
    for g in range(_BPW // _NL):
        sl = pl.ds(g * _NL, _NL)
        idx1_v[sl] = jax.lax.shift_right_logical(idx1_v[sl], 1)
        idx2_v[sl] = jax.lax.shift_right_logical(idx2_v[sl], 1)
    d1 = pltpu.async_copy(t1_hbm.at[idx1_v], r11_v, sem)
    d2 = pltpu.async_copy(t2_hbm.at[idx1_v], r12_v, sem)
    d3 = pltpu.async_copy(t1_hbm.at[idx2_v], r21_v, sem)
    d4 = pltpu.async_copy(t2_hbm.at[idx2_v], r22_v, sem)
    d1.wait()
    d2.wait()
    d3.wait()
    d4.wait()
    pltpu.sync_copy(r11_v, o11.at[pl.ds(base, _BPW)])
    pltpu.sync_copy(r12_v, o12.at[pl.ds(base, _BPW)])
    pltpu.sync_copy(r21_v, o21.at[pl.ds(base, _BPW)])
    pltpu.sync_copy(r22_v, o22.at[pl.ds(base, _BPW)])


_TW = 4096


def _pair_rows(y):
    y3 = y.reshape(_TW // 2, 2, _DIM)
    return jnp.concatenate([y3[:, 0, :], y3[:, 1, :]], axis=1)


def _transpose_body(tt1_ref, tt2_ref, o1_ref, o2_ref):
    o1_ref[...] = _pair_rows(jnp.transpose(tt1_ref[...]))
    o2_ref[...] = _pair_rows(jnp.transpose(tt2_ref[...]))


def _tc_transpose(tt1, tt2):
    grid = (_N + _TW - 1) // _TW
    in_spec = pl.BlockSpec((_DIM, _TW), lambda i: (0, i))
    out_spec = pl.BlockSpec((_TW // 2, 2 * _DIM), lambda i: (i, 0))
    return pl.pallas_call(
        _transpose_body,
        grid=(grid,),
        in_specs=[in_spec, in_spec],
        out_specs=[out_spec, out_spec],
        out_shape=[jax.ShapeDtypeStruct((_N // 2, 2 * _DIM), jnp.float32)] * 2,
    )(tt1, tt2)


def _softplus(x):
    return jnp.logaddexp(x, 0.0)


def _half(ref, parity):
    x = ref[...]
    return jnp.where(parity, x[:, _DIM:], x[:, :_DIM])


def _box_math_body(i1_ref, i2_ref, c1_ref, e1_ref, c2_ref, e2_ref, out_ref):
    p1 = (i1_ref[...] & 1)[:, None] == 1
    p2 = (i2_ref[...] & 1)[:, None] == 1
    c1 = _half(c1_ref, p1)
    w1 = _softplus(_half(e1_ref, p1)) * 0.5
    c2 = _half(c2_ref, p2)
    w2 = _softplus(_half(e2_ref, p2)) * 0.5
    min1 = c1 - w1
    max1 = c1 + w1
    min2 = c2 - w2
    max2 = c2 + w2
    t = _INTERSECTION_TEMP
    meet_min = t * jnp.logaddexp(min1 / t, min2 / t)
    meet_max = -t * jnp.logaddexp(-max1 / t, -max2 / t)
    meet_min = jnp.maximum(meet_min, jnp.maximum(min1, min2))
    meet_max = jnp.minimum(meet_max, jnp.minimum(max1, max2))
    log_overlap = jnp.sum(
        jnp.log(_VOLUME_TEMP * _softplus(
            (meet_max - meet_min - _SOFTPLUS_CONST) / _VOLUME_TEMP) + 1e-20),
        axis=-1)
    log_rhs = jnp.sum(
        jnp.log(_VOLUME_TEMP * _softplus(
            (max2 - min2 - _SOFTPLUS_CONST) / _VOLUME_TEMP) + 1e-20),
        axis=-1)
    out_ref[...] = log_overlap - log_rhs


_TC_BLOCK = 512


def _tc_math(i1, i2, r11, r12, r21, r22):
    grid = _B // _TC_BLOCK
    in_spec = pl.BlockSpec((_TC_BLOCK, 2 * _DIM), lambda i: (i, 0))
    idx_spec = pl.BlockSpec((_TC_BLOCK,), lambda i: (i,))
    return pl.pallas_call(
        _box_math_body,
        grid=(grid,),
        in_specs=[idx_spec, idx_spec] + [in_spec] * 4,
        out_specs=pl.BlockSpec((_TC_BLOCK,), lambda i: (i,)),
        out_shape=jax.ShapeDtypeStruct((_B,), jnp.float32),
    )(i1, i2, r11, r12, r21, r22)


def kernel(idx1, idx2, emb1, emb2, embs1_weight, embs2_weight):
    del emb1, emb2
    i1 = idx1.astype(jnp.int32)
    i2 = idx2.astype(jnp.int32)
    t1p, t2p = _tc_transpose(embs1_weight.T, embs2_weight.T)
    r11, r12, r21, r22 = _sc_gather(t1p, t2p, i1, i2)
    return _tc_math(i1, i2, r11, r12, r21, r22)


# speedup vs baseline: 1.7258x; 1.3362x over previous
import functools

import jax
import jax.numpy as jnp
from jax import lax
from jax.experimental import pallas as pl
from jax.experimental.pallas import tpu as pltpu
from jax.experimental.pallas import tpu_sc as plsc

_B = 4096
_N = 100000
_DIM = 64
_PD = 2 * _DIM
_VOLUME_TEMP = 1.0
_INTERSECTION_TEMP = 0.01
_SOFTPLUS_CONST = 2 * _INTERSECTION_TEMP * 0.5772156649015329

_info = plsc.get_sparse_core_info()
_NC, _NS, _NL = _info.num_cores, _info.num_subcores, _info.num_lanes
_NW = _NC * _NS
_BPW = _B // _NW


_TW = 4096


def _transpose_body(tt1_ref, tt2_ref, o1_ref, o2_ref):
    ipad = (lax.broadcasted_iota(jnp.int32, (_DIM, _PD), 0)
            == lax.broadcasted_iota(jnp.int32, (_DIM, _PD), 1)
            ).astype(jnp.float32)
    dn = (((0,), (0,)), ((), ()))
    o1_ref[...] = lax.dot_general(tt1_ref[...], ipad, dn,
                                  preferred_element_type=jnp.float32)
    o2_ref[...] = lax.dot_general(tt2_ref[...], ipad, dn,
                                  preferred_element_type=jnp.float32)


def _tc_transpose(tt1, tt2):
    grid = (_N + _TW - 1) // _TW
    in_spec = pl.BlockSpec((_DIM, _TW), lambda i: (0, i))
    out_spec = pl.BlockSpec((_TW, _PD), lambda i: (i, 0))
    return pl.pallas_call(
        _transpose_body,
        grid=(grid,),
        in_specs=[in_spec, in_spec],
        out_specs=[out_spec, out_spec],
        out_shape=[jax.ShapeDtypeStruct((_N, _PD), jnp.float32)] * 2,
    )(tt1, tt2)


_sc_mesh = plsc.VectorSubcoreMesh(core_axis_name="c", subcore_axis_name="s")


@functools.partial(
    pl.kernel,
    mesh=_sc_mesh,
    out_type=[jax.ShapeDtypeStruct((_B, _PD), jnp.float32)] * 4,
    scratch_types=[
        pltpu.VMEM((_BPW,), jnp.int32),
        pltpu.VMEM((_BPW,), jnp.int32),
        pltpu.VMEM((_BPW, _PD), jnp.float32),
        pltpu.VMEM((_BPW, _PD), jnp.float32),
        pltpu.VMEM((_BPW, _PD), jnp.float32),
        pltpu.VMEM((_BPW, _PD), jnp.float32),
        pltpu.SemaphoreType.DMA,
    ],
)
def _sc_gather(t1_hbm, t2_hbm, i1_hbm, i2_hbm,
               o11, o12, o21, o22,
               idx1_v, idx2_v, r11_v, r12_v, r21_v, r22_v, sem):
    wid = lax.axis_index("s") * _NC + lax.axis_index("c")
    base = wid * _BPW
    pltpu.sync_copy(i1_hbm.at[pl.ds(base, _BPW)], idx1_v)
    pltpu.sync_copy(i2_hbm.at[pl.ds(base, _BPW)], idx2_v)
    d1 = pltpu.async_copy(t1_hbm.at[idx1_v], r11_v, sem)
    d2 = pltpu.async_copy(t2_hbm.at[idx1_v], r12_v, sem)
    d3 = pltpu.async_copy(t1_hbm.at[idx2_v], r21_v, sem)
    d4 = pltpu.async_copy(t2_hbm.at[idx2_v], r22_v, sem)
    d1.wait()
    d2.wait()
    d3.wait()
    d4.wait()
    pltpu.sync_copy(r11_v, o11.at[pl.ds(base, _BPW)])
    pltpu.sync_copy(r12_v, o12.at[pl.ds(base, _BPW)])
    pltpu.sync_copy(r21_v, o21.at[pl.ds(base, _BPW)])
    pltpu.sync_copy(r22_v, o22.at[pl.ds(base, _BPW)])


def _softplus(x):
    return jnp.logaddexp(x, 0.0)


def _box_math_body(c1_ref, e1_ref, c2_ref, e2_ref, out_ref):
    c1 = c1_ref[:, :_DIM]
    w1 = _softplus(e1_ref[:, :_DIM]) * 0.5
    c2 = c2_ref[:, :_DIM]
    w2 = _softplus(e2_ref[:, :_DIM]) * 0.5
    min1 = c1 - w1
    max1 = c1 + w1
    min2 = c2 - w2
    max2 = c2 + w2
    t = _INTERSECTION_TEMP
    meet_min = t * jnp.logaddexp(min1 / t, min2 / t)
    meet_max = -t * jnp.logaddexp(-max1 / t, -max2 / t)
    meet_min = jnp.maximum(meet_min, jnp.maximum(min1, min2))
    meet_max = jnp.minimum(meet_max, jnp.minimum(max1, max2))
    log_overlap = jnp.sum(
        jnp.log(_VOLUME_TEMP * _softplus(
            (meet_max - meet_min - _SOFTPLUS_CONST) / _VOLUME_TEMP) + 1e-20),
        axis=-1)
    log_rhs = jnp.sum(
        jnp.log(_VOLUME_TEMP * _softplus(
            (max2 - min2 - _SOFTPLUS_CONST) / _VOLUME_TEMP) + 1e-20),
        axis=-1)
    out_ref[...] = log_overlap - log_rhs


_TC_BLOCK = 512


def _tc_math(r11, r12, r21, r22):
    grid = _B // _TC_BLOCK
    in_spec = pl.BlockSpec((_TC_BLOCK, _PD), lambda i: (i, 0))
    return pl.pallas_call(
        _box_math_body,
        grid=(grid,),
        in_specs=[in_spec] * 4,
        out_specs=pl.BlockSpec((_TC_BLOCK,), lambda i: (i,)),
        out_shape=jax.ShapeDtypeStruct((_B,), jnp.float32),
    )(r11, r12, r21, r22)


def kernel(idx1, idx2, emb1, emb2, embs1_weight, embs2_weight):
    del emb1, emb2
    i1 = idx1.astype(jnp.int32)
    i2 = idx2.astype(jnp.int32)
    t1p, t2p = _tc_transpose(embs1_weight.T, embs2_weight.T)
    r11, r12, r21, r22 = _sc_gather(t1p, t2p, i1, i2)
    return _tc_math(r11, r12, r21, r22)


# speedup vs baseline: 1.9458x; 1.1275x over previous
import functools

import jax
import jax.numpy as jnp
from jax import lax
from jax.experimental import pallas as pl
from jax.experimental.pallas import tpu as pltpu
from jax.experimental.pallas import tpu_sc as plsc

_B = 4096
_N = 100000
_DIM = 64
_PD = 2 * _DIM
_VOLUME_TEMP = 1.0
_INTERSECTION_TEMP = 0.01
_SOFTPLUS_CONST = 2 * _INTERSECTION_TEMP * 0.5772156649015329

_info = plsc.get_sparse_core_info()
_NC, _NS, _NL = _info.num_cores, _info.num_subcores, _info.num_lanes
_NW = _NC * _NS
_BPW = _B // _NW


_TW = 4096


def _transpose_body(tt1_ref, tt2_ref, o_ref):
    row = lax.broadcasted_iota(jnp.int32, (_DIM, _PD), 0)
    col = lax.broadcasted_iota(jnp.int32, (_DIM, _PD), 1)
    i_low = (row == col).astype(jnp.float32)
    i_high = (row + _DIM == col).astype(jnp.float32)
    dn = (((0,), (0,)), ((), ()))
    o_ref[...] = (
        lax.dot_general(tt1_ref[...], i_low, dn,
                        preferred_element_type=jnp.float32)
        + lax.dot_general(tt2_ref[...], i_high, dn,
                          preferred_element_type=jnp.float32))


def _tc_transpose(tt1, tt2):
    grid = (_N + _TW - 1) // _TW
    in_spec = pl.BlockSpec((_DIM, _TW), lambda i: (0, i))
    return pl.pallas_call(
        _transpose_body,
        grid=(grid,),
        in_specs=[in_spec, in_spec],
        out_specs=pl.BlockSpec((_TW, _PD), lambda i: (i, 0)),
        out_shape=jax.ShapeDtypeStruct((_N, _PD), jnp.float32),
    )(tt1, tt2)


_sc_mesh = plsc.VectorSubcoreMesh(core_axis_name="c", subcore_axis_name="s")


@functools.partial(
    pl.kernel,
    mesh=_sc_mesh,
    out_type=[jax.ShapeDtypeStruct((_B, _PD), jnp.float32)] * 2,
    scratch_types=[
        pltpu.VMEM((_BPW,), jnp.int32),
        pltpu.VMEM((_BPW,), jnp.int32),
        pltpu.VMEM((_BPW, _PD), jnp.float32),
        pltpu.VMEM((_BPW, _PD), jnp.float32),
        pltpu.SemaphoreType.DMA,
    ],
)
def _sc_gather(t_hbm, i1_hbm, i2_hbm,
               o1, o2,
               idx1_v, idx2_v, r1_v, r2_v, sem):
    wid = lax.axis_index("s") * _NC + lax.axis_index("c")
    base = wid * _BPW
    pltpu.sync_copy(i1_hbm.at[pl.ds(base, _BPW)], idx1_v)
    pltpu.sync_copy(i2_hbm.at[pl.ds(base, _BPW)], idx2_v)
    d1 = pltpu.async_copy(t_hbm.at[idx1_v], r1_v, sem)
    d2 = pltpu.async_copy(t_hbm.at[idx2_v], r2_v, sem)
    d1.wait()
    d2.wait()
    pltpu.sync_copy(r1_v, o1.at[pl.ds(base, _BPW)])
    pltpu.sync_copy(r2_v, o2.at[pl.ds(base, _BPW)])


def _softplus(x):
    return jnp.logaddexp(x, 0.0)


def _box_math_body(b1_ref, b2_ref, out_ref):
    c1 = b1_ref[:, :_DIM]
    w1 = _softplus(b1_ref[:, _DIM:]) * 0.5
    c2 = b2_ref[:, :_DIM]
    w2 = _softplus(b2_ref[:, _DIM:]) * 0.5
    min1 = c1 - w1
    max1 = c1 + w1
    min2 = c2 - w2
    max2 = c2 + w2
    t = _INTERSECTION_TEMP
    meet_min = t * jnp.logaddexp(min1 / t, min2 / t)
    meet_max = -t * jnp.logaddexp(-max1 / t, -max2 / t)
    meet_min = jnp.maximum(meet_min, jnp.maximum(min1, min2))
    meet_max = jnp.minimum(meet_max, jnp.minimum(max1, max2))
    log_overlap = jnp.sum(
        jnp.log(_VOLUME_TEMP * _softplus(
            (meet_max - meet_min - _SOFTPLUS_CONST) / _VOLUME_TEMP) + 1e-20),
        axis=-1)
    log_rhs = jnp.sum(
        jnp.log(_VOLUME_TEMP * _softplus(
            (max2 - min2 - _SOFTPLUS_CONST) / _VOLUME_TEMP) + 1e-20),
        axis=-1)
    out_ref[...] = log_overlap - log_rhs


_TC_BLOCK = 512


def _tc_math(r1, r2):
    grid = _B // _TC_BLOCK
    in_spec = pl.BlockSpec((_TC_BLOCK, _PD), lambda i: (i, 0))
    return pl.pallas_call(
        _box_math_body,
        grid=(grid,),
        in_specs=[in_spec, in_spec],
        out_specs=pl.BlockSpec((_TC_BLOCK,), lambda i: (i,)),
        out_shape=jax.ShapeDtypeStruct((_B,), jnp.float32),
    )(r1, r2)


def kernel(idx1, idx2, emb1, emb2, embs1_weight, embs2_weight):
    del emb1, emb2
    i1 = idx1.astype(jnp.int32)
    i2 = idx2.astype(jnp.int32)
    tp = _tc_transpose(embs1_weight.T, embs2_weight.T)
    r1, r2 = _sc_gather(tp, i1, i2)
    return _tc_math(r1, r2)


# speedup vs baseline: 2.1174x; 1.0882x over previous
import functools

import jax
import jax.numpy as jnp
from jax import lax
from jax.experimental import pallas as pl
from jax.experimental.pallas import tpu as pltpu
from jax.experimental.pallas import tpu_sc as plsc

_B = 4096
_N = 100000
_DIM = 64
_PD = 2 * _DIM
_VOLUME_TEMP = 1.0
_INTERSECTION_TEMP = 0.01
_SOFTPLUS_CONST = 2 * _INTERSECTION_TEMP * 0.5772156649015329

_info = plsc.get_sparse_core_info()
_NC, _NS, _NL = _info.num_cores, _info.num_subcores, _info.num_lanes
_NW = _NC * _NS
_BPW = _B // _NW


_TW = 8192


def _transpose_body(tt1_ref, tt2_ref, o_ref):
    row = lax.broadcasted_iota(jnp.int32, (_DIM, _PD), 0)
    col = lax.broadcasted_iota(jnp.int32, (_DIM, _PD), 1)
    i_low = (row == col).astype(jnp.float32)
    i_high = (row + _DIM == col).astype(jnp.float32)
    dn = (((0,), (0,)), ((), ()))
    o_ref[...] = (
        lax.dot_general(tt1_ref[...], i_low, dn,
                        preferred_element_type=jnp.float32)
        + lax.dot_general(tt2_ref[...], i_high, dn,
                          preferred_element_type=jnp.float32))


def _tc_transpose(tt1, tt2):
    grid = (_N + _TW - 1) // _TW
    in_spec = pl.BlockSpec((_DIM, _TW), lambda i: (0, i))
    return pl.pallas_call(
        _transpose_body,
        grid=(grid,),
        in_specs=[in_spec, in_spec],
        out_specs=pl.BlockSpec((_TW, _PD), lambda i: (i, 0)),
        out_shape=jax.ShapeDtypeStruct((_N, _PD), jnp.float32),
    )(tt1, tt2)


_sc_mesh = plsc.VectorSubcoreMesh(core_axis_name="c", subcore_axis_name="s")


@functools.partial(
    pl.kernel,
    mesh=_sc_mesh,
    out_type=[jax.ShapeDtypeStruct((_B, _PD), jnp.float32)] * 2,
    scratch_types=[
        pltpu.VMEM((_BPW,), jnp.int32),
        pltpu.VMEM((_BPW,), jnp.int32),
        pltpu.VMEM((_BPW, _PD), jnp.float32),
        pltpu.VMEM((_BPW, _PD), jnp.float32),
        pltpu.SemaphoreType.DMA,
    ],
)
def _sc_gather(t_hbm, i1_hbm, i2_hbm,
               o1, o2,
               idx1_v, idx2_v, r1_v, r2_v, sem):
    wid = lax.axis_index("s") * _NC + lax.axis_index("c")
    base = wid * _BPW
    pltpu.sync_copy(i1_hbm.at[pl.ds(base, _BPW)], idx1_v)
    pltpu.sync_copy(i2_hbm.at[pl.ds(base, _BPW)], idx2_v)
    d1 = pltpu.async_copy(t_hbm.at[idx1_v], r1_v, sem)
    d2 = pltpu.async_copy(t_hbm.at[idx2_v], r2_v, sem)
    d1.wait()
    d2.wait()
    pltpu.sync_copy(r1_v, o1.at[pl.ds(base, _BPW)])
    pltpu.sync_copy(r2_v, o2.at[pl.ds(base, _BPW)])


def _softplus(x):
    return jnp.logaddexp(x, 0.0)


def _box_math_body(b1_ref, b2_ref, out_ref):
    c1 = b1_ref[:, :_DIM]
    w1 = _softplus(b1_ref[:, _DIM:]) * 0.5
    c2 = b2_ref[:, :_DIM]
    w2 = _softplus(b2_ref[:, _DIM:]) * 0.5
    min1 = c1 - w1
    max1 = c1 + w1
    min2 = c2 - w2
    max2 = c2 + w2
    t = _INTERSECTION_TEMP
    meet_min = t * jnp.logaddexp(min1 / t, min2 / t)
    meet_max = -t * jnp.logaddexp(-max1 / t, -max2 / t)
    meet_min = jnp.maximum(meet_min, jnp.maximum(min1, min2))
    meet_max = jnp.minimum(meet_max, jnp.minimum(max1, max2))
    log_overlap = jnp.sum(
        jnp.log(_VOLUME_TEMP * _softplus(
            (meet_max - meet_min - _SOFTPLUS_CONST) / _VOLUME_TEMP) + 1e-20),
        axis=-1)
    log_rhs = jnp.sum(
        jnp.log(_VOLUME_TEMP * _softplus(
            (max2 - min2 - _SOFTPLUS_CONST) / _VOLUME_TEMP) + 1e-20),
        axis=-1)
    out_ref[...] = log_overlap - log_rhs


_TC_BLOCK = 512


def _tc_math(r1, r2):
    grid = _B // _TC_BLOCK
    in_spec = pl.BlockSpec((_TC_BLOCK, _PD), lambda i: (i, 0))
    return pl.pallas_call(
        _box_math_body,
        grid=(grid,),
        in_specs=[in_spec, in_spec],
        out_specs=pl.BlockSpec((_TC_BLOCK,), lambda i: (i,)),
        out_shape=jax.ShapeDtypeStruct((_B,), jnp.float32),
    )(r1, r2)


def kernel(idx1, idx2, emb1, emb2, embs1_weight, embs2_weight):
    del emb1, emb2
    i1 = idx1.astype(jnp.int32)
    i2 = idx2.astype(jnp.int32)
    tp = _tc_transpose(embs1_weight.T, embs2_weight.T)
    r1, r2 = _sc_gather(tp, i1, i2)
    return _tc_math(r1, r2)
